# baseline (speedup 1.0000x reference)
import functools

import jax
import jax.numpy as jnp
from jax import lax
from jax.experimental import pallas as pl
from jax.experimental.pallas import tpu as pltpu
from jax.experimental.pallas import tpu_sc as plsc

_NC = 2
_NS = 16
_L = 128
_K = 8


def _sc_gather(tab, idx2d):
    nblk, L = idx2d.shape
    D = tab.shape[1]
    nw = _NC * _NS
    nblk_per_w = nblk // nw
    assert nblk_per_w * nw == nblk
    K = _K
    nchunks = nblk_per_w // K
    assert nchunks * K == nblk_per_w

    mesh = plsc.VectorSubcoreMesh(core_axis_name="c", subcore_axis_name="s")

    @functools.partial(
        pl.kernel,
        out_type=jax.ShapeDtypeStruct((nblk, L, D), jnp.float32),
        mesh=mesh,
        scratch_types=[
            pltpu.VMEM((K, L), jnp.int32),
            pltpu.VMEM((K, L, D), jnp.float32),
            pltpu.SemaphoreType.DMA,
        ],
        compiler_params=pltpu.CompilerParams(use_tc_tiling_on_sc=False),
    )
    def gather_kernel(tab_hbm, idx_hbm, out_hbm, idx_v, rows_v, sem):
        wid = lax.axis_index("s") * _NC + lax.axis_index("c")
        base = wid * nblk_per_w

        def chunk(c, carry):
            blk0 = base + c * K
            pltpu.sync_copy(idx_hbm.at[pl.ds(blk0, K)], idx_v)
            cps = [
                pltpu.async_copy(tab_hbm.at[idx_v.at[j]], rows_v.at[j], sem)
                for j in range(K)
            ]
            for cp in cps:
                cp.wait()
            pltpu.sync_copy(rows_v, out_hbm.at[pl.ds(blk0, K)])
            return carry

        lax.fori_loop(0, nchunks, chunk, 0)

    return gather_kernel(tab, idx2d)


def _towers_block(e, fm, w1, b1, w2, b2, w3, b3):
    h = jnp.maximum(
        jnp.dot(e, w1, preferred_element_type=jnp.float32) + b1[None, :], 0.0)
    h = jnp.maximum(
        jnp.dot(h, w2, preferred_element_type=jnp.float32) + b2[None, :], 0.0)
    deep = jnp.dot(h, w3, preferred_element_type=jnp.float32) + b3[None, :]
    z = deep + fm
    return 1.0 / (1.0 + jnp.exp(-z))


def _tc_towers(emb, smat, params, block_b):
    Bn, din = emb.shape

    def body(emb_ref, smat_ref,
             cw1, cb1, cw2, cb2, cw3, cb3,
             tw1, tb1, tw2, tb2, tw3, tb3, out_ref):
        e = emb_ref[...]
        s = jnp.dot(e, smat_ref[...], preferred_element_type=jnp.float32)
        ss = jnp.sum(s * s, axis=1, keepdims=True)
        sq = jnp.sum(e * e, axis=1, keepdims=True)
        fm = 0.5 * (ss - sq)
        cvr = _towers_block(e, fm, cw1[...], cb1[...], cw2[...], cb2[...],
                            cw3[...], cb3[...])
        ctr = _towers_block(e, fm, tw1[...], tb1[...], tw2[...], tb2[...],
                            tw3[...], tb3[...])
        res = jnp.concatenate([cvr, ctr, cvr * ctr], axis=1)
        out_ref[...] = jnp.clip(res, 1e-15, 1.0 - 1e-15)

    full = lambda shape: pl.BlockSpec(shape, lambda i: (0,) * len(shape))
    in_specs = [pl.BlockSpec((block_b, din), lambda i: (i, 0)),
                full(smat.shape)]
    in_specs += [full(p.shape) for p in params]

    return pl.pallas_call(
        body,
        grid=(Bn // block_b,),
        in_specs=in_specs,
        out_specs=pl.BlockSpec((block_b, 3), lambda i: (i, 0)),
        out_shape=jax.ShapeDtypeStruct((Bn, 3), jnp.float32),
    )(emb, smat, *params)


def kernel(x, tables, cvr_w1, cvr_b1, cvr_w2, cvr_b2, cvr_w3, cvr_b3,
           ctr_w1, ctr_b1, ctr_w2, ctr_b2, ctr_w3, ctr_b3):
    F, V, D = tables.shape
    B = x.shape[0]
    flat_idx = (x + (jnp.arange(F, dtype=jnp.int32) * V)[None, :]).reshape(-1)
    idx2d = flat_idx.reshape(-1, _L)
    tab = tables.reshape(F * V, D)
    emb = _sc_gather(tab, idx2d)
    emb_flat = emb.reshape(B, F * D)
    smat = jnp.tile(jnp.eye(D, dtype=jnp.float32), (F, 1))
    params = (cvr_w1, cvr_b1, cvr_w2, cvr_b2, cvr_w3, cvr_b3,
              ctr_w1, ctr_b1, ctr_w2, ctr_b2, ctr_w3, ctr_b3)
    return _tc_towers(emb_flat, smat, params, block_b=512)
